# baseline (speedup 1.0000x reference)
import functools

import numpy as np
import jax
import jax.numpy as jnp
from jax import lax
from jax.experimental import pallas as pl
from jax.experimental.pallas import tpu as pltpu
from jax.experimental.pallas import tpu_sc as plsc

NC = 2
NS = 16
NW = NC * NS
B = 128
NQ = 4


def _cdiv(a, b):
    return (a + b - 1) // b


def _pack_perm(d):
    perm = np.empty((d,), np.int32)
    for m in range(d // 32):
        for i in range(16):
            perm[32 * m + 2 * i] = 32 * m + i
            perm[32 * m + 2 * i + 1] = 32 * m + 16 + i
    return perm


def _zero_vmem(ref, rows, cols):
    z = jnp.zeros((16,), jnp.float32)

    def body(r, carry):
        for m in range(cols // 16):
            ref[r, pl.ds(16 * m, 16)] = z
        return carry

    lax.fori_loop(0, rows, body, 0)


def _sc_degrees(edx, n_pad):
    nb = edx.shape[1]
    rpt = n_pad // NS
    hpt = rpt // 2
    mesh = plsc.VectorSubcoreMesh(core_axis_name="c", subcore_axis_name="s")

    @functools.partial(
        pl.kernel,
        out_type=jax.ShapeDtypeStruct((NC * 2 * n_pad,), jnp.float32),
        mesh=mesh,
        scratch_types=[
            pltpu.VMEM((nb, 2, B), jnp.int32),
            pltpu.VMEM((B,), jnp.float32),
            pltpu.VMEM((hpt,), jnp.float32),
            pltpu.VMEM_SHARED((n_pad,), jnp.float32),
            pltpu.VMEM_SHARED((n_pad,), jnp.float32),
        ],
        compiler_params=pltpu.CompilerParams(use_tc_tiling_on_sc=False),
    )
    def k(edx_h, out_h, exv, ones_v, dr_v, dego_sh, degi_sh):
        c = lax.axis_index("c")
        s = lax.axis_index("s")
        wid = s * NC + c
        r0 = s * rpt
        def zb(r, carry):
            dr_v[pl.ds(16 * r, 16)] = jnp.zeros((16,), jnp.float32)
            return carry

        lax.fori_loop(0, hpt // 16, zb, 0)
        for i in range(B // 16):
            ones_v[pl.ds(i * 16, 16)] = jnp.ones((16,), jnp.float32)
        for sh in (dego_sh, degi_sh):
            for half in range(2):
                pltpu.sync_copy(dr_v, sh.at[pl.ds(r0 + half * hpt, hpt)])
        pltpu.sync_copy(edx_h.at[wid], exv)
        plsc.subcore_barrier()

        def body(j, carry):
            pltpu.sync_copy(ones_v, dego_sh.at[exv.at[j, 0]], add=True)
            pltpu.sync_copy(ones_v, degi_sh.at[exv.at[j, 1]], add=True)
            return carry

        lax.fori_loop(0, nb, body, 0)
        plsc.subcore_barrier()
        for t, sh in enumerate((dego_sh, degi_sh)):
            base = (c * 2 + t) * n_pad + r0
            for half in range(2):
                pltpu.sync_copy(sh.at[pl.ds(r0 + half * hpt, hpt)], dr_v)
                pltpu.sync_copy(dr_v, out_h.at[pl.ds(base + half * hpt, hpt)])

    return k(edx)


def _sc_aggregate(h_bf, edx, n_pad, d):
    nb = edx.shape[1]
    rpt = n_pad // NS
    qpt = rpt // NQ
    mesh = plsc.VectorSubcoreMesh(core_axis_name="c", subcore_axis_name="s")

    @functools.partial(
        pl.kernel,
        out_type=jax.ShapeDtypeStruct((NC, n_pad, d), jnp.bfloat16),
        mesh=mesh,
        scratch_types=[
            pltpu.VMEM((nb, 2, B), jnp.int32),
            pltpu.VMEM((B, d), jnp.bfloat16),
            pltpu.VMEM((B, d), jnp.float32),
            pltpu.VMEM((B, d), jnp.float32),
            pltpu.VMEM((qpt, d), jnp.float32),
            pltpu.VMEM((qpt, d), jnp.bfloat16),
            pltpu.VMEM_SHARED((n_pad, d), jnp.bfloat16),
            pltpu.VMEM_SHARED((n_pad, d), jnp.float32),
            pltpu.SemaphoreType.DMA,
        ],
        compiler_params=pltpu.CompilerParams(
            use_tc_tiling_on_sc=False,
            needs_layout_passes=False,
        ),
    )
    def k(h_h, edx_h, out_h, exv, rbf_v, rows0_v, rows1_v, dr_v, drb_v,
          tab_sh, acc_sh, sem):
        c = lax.axis_index("c")
        s = lax.axis_index("s")
        wid = s * NC + c
        r0 = s * rpt
        for q in range(NQ):
            rq = r0 + q * qpt
            pltpu.sync_copy(h_h.at[pl.ds(rq, qpt)], drb_v)
            pltpu.sync_copy(drb_v, tab_sh.at[pl.ds(rq, qpt)])
        _zero_vmem(dr_v, qpt, d)
        for q in range(NQ):
            pltpu.sync_copy(dr_v, acc_sh.at[pl.ds(r0 + q * qpt, qpt)])
        pltpu.sync_copy(edx_h.at[wid], exv)
        plsc.subcore_barrier()

        def gather_unpack(j, rows_v):
            pltpu.sync_copy(tab_sh.at[exv.at[j, 0]], rbf_v)

            def conv(r, carry):
                for m in range(d // 32):
                    u = rbf_v[r, pl.ds(32 * m, 32)]
                    a, b = plsc.unpack(u, format=plsc.PackFormat.INTERLEAVED)
                    rows_v[r, pl.ds(32 * m, 16)] = a
                    rows_v[r, pl.ds(32 * m + 16, 16)] = b
                return carry

            lax.fori_loop(0, B, conv, 0)

        gather_unpack(0, rows0_v)
        pltpu.async_copy(rows0_v, acc_sh.at[exv.at[0, 1]], sem, add=True)

        def body(jj, carry):
            j = jj * 2 + 1
            gather_unpack(j, rows1_v)
            pltpu.make_async_copy(
                rows0_v, acc_sh.at[exv.at[j - 1, 1]], sem).wait()
            pltpu.async_copy(rows1_v, acc_sh.at[exv.at[j, 1]], sem, add=True)
            gather_unpack(j + 1, rows0_v)
            pltpu.make_async_copy(
                rows1_v, acc_sh.at[exv.at[j, 1]], sem).wait()
            pltpu.async_copy(rows0_v, acc_sh.at[exv.at[j + 1, 1]], sem,
                             add=True)
            return carry

        lax.fori_loop(0, (nb - 1) // 2, body, 0)
        last = ((nb - 1) // 2) * 2
        if nb % 2 == 0:
            j = nb - 1
            gather_unpack(j, rows1_v)
            pltpu.make_async_copy(
                rows0_v, acc_sh.at[exv.at[last, 1]], sem).wait()
            pltpu.async_copy(rows1_v, acc_sh.at[exv.at[j, 1]], sem, add=True)
            pltpu.make_async_copy(
                rows1_v, acc_sh.at[exv.at[j, 1]], sem).wait()
        else:
            pltpu.make_async_copy(
                rows0_v, acc_sh.at[exv.at[last, 1]], sem).wait()
        plsc.subcore_barrier()
        for q in range(NQ):
            rq = r0 + q * qpt
            pltpu.sync_copy(acc_sh.at[pl.ds(rq, qpt)], dr_v)

            def conv(r, carry):
                for m in range(d // 32):
                    a = dr_v[r, pl.ds(32 * m, 16)]
                    b = dr_v[r, pl.ds(32 * m + 16, 16)]
                    drb_v[r, pl.ds(32 * m, 32)] = plsc.pack(
                        a, b, format=plsc.PackFormat.INTERLEAVED)
                return carry

            lax.fori_loop(0, qpt, conv, 0)
            pltpu.sync_copy(drb_v, out_h.at[c, pl.ds(rq, qpt)])

    return k(h_bf, edx)


def _tc_stage1(x_pad, w1, dp4, n_pad, hid):
    def body(x_ref, w_ref, dp_ref, ha_ref, hb_ref, dego_ref, degi_ref):
        do_ = jnp.maximum(dp_ref[0, 0] + dp_ref[1, 0], 1.0)
        di_ = jnp.maximum(dp_ref[0, 1] + dp_ref[1, 1], 1.0)
        dego = lax.rsqrt(do_)
        degi = lax.rsqrt(di_)
        dego_ref[...] = dego
        degi_ref[...] = degi
        h = jnp.dot(x_ref[...] * dego, w_ref[...],
                    preferred_element_type=jnp.float32)
        ha_ref[...] = h[:, : hid // 2]
        hb_ref[...] = h[:, hid // 2:]

    return pl.pallas_call(
        body,
        out_shape=(
            jax.ShapeDtypeStruct((n_pad, hid // 2), jnp.float32),
            jax.ShapeDtypeStruct((n_pad, hid // 2), jnp.float32),
            jax.ShapeDtypeStruct((n_pad, 1), jnp.float32),
            jax.ShapeDtypeStruct((n_pad, 1), jnp.float32),
        ),
    )(x_pad, w1, dp4)


def _tc_stage2(a1a, a1b, dego, degi, w2p, n_pad, hid, hid2):
    def body(aa_ref, ab_ref, dego_ref, degi_ref, w_ref, m_ref):
        def half(ref):
            a = ref[0].astype(jnp.float32) + ref[1].astype(jnp.float32)
            h = a * degi_ref[...]
            h = jnp.where(h >= 0, h, 0.01 * h)
            return h * dego_ref[...]

        ha = half(aa_ref)
        hb = half(ab_ref)
        m_ref[...] = (
            jnp.dot(ha, w_ref[: hid // 2], preferred_element_type=jnp.float32)
            + jnp.dot(hb, w_ref[hid // 2:], preferred_element_type=jnp.float32)
        )

    return pl.pallas_call(
        body,
        out_shape=jax.ShapeDtypeStruct((n_pad, hid2), jnp.float32),
    )(a1a, a1b, dego, degi, w2p)


def _tc_stage3(a2, degi, wlp, bl, wc, n):
    def body(a_ref, degi_ref, wl_ref, bl_ref, wc_ref, o_ref):
        a = a_ref[0].astype(jnp.float32) + a_ref[1].astype(jnp.float32)
        h = a * degi_ref[...]
        h = jnp.where(h >= 0, h, 0.01 * h)
        s = jnp.sum(h, axis=0, keepdims=True) * jnp.float32(1.0 / n)
        p = lax.dot_general(s, wl_ref[...], (((1,), (1,)), ((), ())),
                            preferred_element_type=jnp.float32) + bl_ref[...]
        o_ref[...] = lax.dot_general(p, wc_ref[...], (((1,), (1,)), ((), ())),
                                     preferred_element_type=jnp.float32)

    return pl.pallas_call(
        body,
        out_shape=jax.ShapeDtypeStruct((1, wc.shape[0]), jnp.float32),
    )(a2, degi, wlp, bl, wc)


def kernel(features, edge_index, W1, W2, Wl, bl, Wc):
    n, din = features.shape
    e = edge_index.shape[1]
    hid = W1.shape[1]
    hid2 = W2.shape[1]
    n_pad = _cdiv(n, NS * 16) * NS * 16
    nb = _cdiv(e, NW * B)
    e_pad = NW * nb * B

    src = edge_index[0].astype(jnp.int32)
    dst = edge_index[1].astype(jnp.int32)
    padi = jnp.full((e_pad - e,), n, jnp.int32)
    src3 = jnp.concatenate([src, padi]).reshape(NW, nb, B)
    dst3 = jnp.concatenate([dst, padi]).reshape(NW, nb, B)
    edx = jnp.stack([src3, dst3], axis=2)
    x_pad = jnp.concatenate(
        [features, jnp.zeros((n_pad - n, din), jnp.float32)])
    p64 = jnp.asarray(_pack_perm(hid2))
    w2p = W2[jnp.concatenate([p64, hid2 + p64])]
    wlp = Wl[:, p64]

    dp = _sc_degrees(edx, n_pad).reshape(NC, 2, n_pad, 1)
    h1a, h1b, dego, degi = _tc_stage1(x_pad, W1, dp, n_pad, hid)
    a1a = _sc_aggregate(h1a[:, p64].astype(jnp.bfloat16), edx, n_pad, hid2)
    a1b = _sc_aggregate(h1b[:, p64].astype(jnp.bfloat16), edx, n_pad, hid2)
    m2 = _tc_stage2(a1a, a1b, dego, degi, w2p, n_pad, hid, hid2)
    a2 = _sc_aggregate(m2[:, p64].astype(jnp.bfloat16), edx, n_pad, hid2)
    return _tc_stage3(a2, degi, wlp, bl, Wc, n)


# speedup vs baseline: 1.4117x; 1.4117x over previous
import functools

import numpy as np
import jax
import jax.numpy as jnp
from jax import lax
from jax.experimental import pallas as pl
from jax.experimental.pallas import tpu as pltpu
from jax.experimental.pallas import tpu_sc as plsc

NC = 2
NS = 16
NW = NC * NS
B = 128
NQ = 4


def _cdiv(a, b):
    return (a + b - 1) // b


def _pack_perm(d):
    perm = np.empty((d,), np.int32)
    for m in range(d // 32):
        for i in range(16):
            perm[32 * m + 2 * i] = 32 * m + i
            perm[32 * m + 2 * i + 1] = 32 * m + 16 + i
    return perm


def _zero_vmem(ref, rows, cols):
    z = jnp.zeros((16,), jnp.float32)

    def body(r, carry):
        for m in range(cols // 16):
            ref[r, pl.ds(16 * m, 16)] = z
        return carry

    lax.fori_loop(0, rows, body, 0)


def _sc_degrees(edx, n_pad):
    nb = edx.shape[1]
    rpt = n_pad // NS
    hpt = rpt // 2
    mesh = plsc.VectorSubcoreMesh(core_axis_name="c", subcore_axis_name="s")

    @functools.partial(
        pl.kernel,
        out_type=jax.ShapeDtypeStruct((NC * 2 * n_pad,), jnp.float32),
        mesh=mesh,
        scratch_types=[
            pltpu.VMEM((nb, 2, B), jnp.int32),
            pltpu.VMEM((B,), jnp.float32),
            pltpu.VMEM((hpt,), jnp.float32),
            pltpu.VMEM_SHARED((n_pad,), jnp.float32),
            pltpu.VMEM_SHARED((n_pad,), jnp.float32),
        ],
        compiler_params=pltpu.CompilerParams(use_tc_tiling_on_sc=False),
    )
    def k(edx_h, out_h, exv, ones_v, dr_v, dego_sh, degi_sh):
        c = lax.axis_index("c")
        s = lax.axis_index("s")
        wid = s * NC + c
        r0 = s * rpt
        def zb(r, carry):
            dr_v[pl.ds(16 * r, 16)] = jnp.zeros((16,), jnp.float32)
            return carry

        lax.fori_loop(0, hpt // 16, zb, 0)
        for i in range(B // 16):
            ones_v[pl.ds(i * 16, 16)] = jnp.ones((16,), jnp.float32)
        for sh in (dego_sh, degi_sh):
            for half in range(2):
                pltpu.sync_copy(dr_v, sh.at[pl.ds(r0 + half * hpt, hpt)])
        pltpu.sync_copy(edx_h.at[wid], exv)
        plsc.subcore_barrier()

        def body(j, carry):
            pltpu.sync_copy(ones_v, dego_sh.at[exv.at[j, 0]], add=True)
            pltpu.sync_copy(ones_v, degi_sh.at[exv.at[j, 1]], add=True)
            return carry

        lax.fori_loop(0, nb, body, 0)
        plsc.subcore_barrier()
        for t, sh in enumerate((dego_sh, degi_sh)):
            base = (c * 2 + t) * n_pad + r0
            for half in range(2):
                pltpu.sync_copy(sh.at[pl.ds(r0 + half * hpt, hpt)], dr_v)
                pltpu.sync_copy(dr_v, out_h.at[pl.ds(base + half * hpt, hpt)])

    return k(edx)


def _sc_aggregate(h_bf, edx, n_pad, d):
    nb = edx.shape[1]
    rpt = n_pad // NS
    qpt = rpt // NQ
    mesh = plsc.VectorSubcoreMesh(core_axis_name="c", subcore_axis_name="s")

    @functools.partial(
        pl.kernel,
        out_type=jax.ShapeDtypeStruct((NC, n_pad, d), jnp.bfloat16),
        mesh=mesh,
        scratch_types=[
            pltpu.VMEM((nb, 2, B), jnp.int32),
            pltpu.VMEM((B, d), jnp.bfloat16),
            pltpu.VMEM((B, d), jnp.float32),
            pltpu.VMEM((B, d), jnp.float32),
            pltpu.VMEM((qpt, d), jnp.float32),
            pltpu.VMEM((qpt, d), jnp.bfloat16),
            pltpu.VMEM_SHARED((n_pad, d), jnp.bfloat16),
            pltpu.VMEM_SHARED((n_pad, d), jnp.float32),
            pltpu.SemaphoreType.DMA,
        ],
        compiler_params=pltpu.CompilerParams(
            use_tc_tiling_on_sc=False,
            needs_layout_passes=False,
        ),
    )
    def k(h_h, edx_h, out_h, exv, rbf_v, rows0_v, rows1_v, dr_v, drb_v,
          tab_sh, acc_sh, sem):
        c = lax.axis_index("c")
        s = lax.axis_index("s")
        wid = s * NC + c
        r0 = s * rpt
        for q in range(NQ):
            rq = r0 + q * qpt
            pltpu.sync_copy(h_h.at[pl.ds(rq, qpt)], drb_v)
            pltpu.sync_copy(drb_v, tab_sh.at[pl.ds(rq, qpt)])
        _zero_vmem(dr_v, qpt, d)
        for q in range(NQ):
            pltpu.sync_copy(dr_v, acc_sh.at[pl.ds(r0 + q * qpt, qpt)])
        pltpu.sync_copy(edx_h.at[wid], exv)
        plsc.subcore_barrier()

        def gather_unpack(j, rows_v):
            pltpu.sync_copy(tab_sh.at[exv.at[j, 0]], rbf_v)

            @plsc.parallel_loop(0, B, unroll=8)
            def conv(r):
                for m in range(d // 32):
                    u = rbf_v[r, pl.ds(32 * m, 32)]
                    a, b = plsc.unpack(u, format=plsc.PackFormat.INTERLEAVED)
                    rows_v[r, pl.ds(32 * m, 16)] = a
                    rows_v[r, pl.ds(32 * m + 16, 16)] = b

        gather_unpack(0, rows0_v)
        pltpu.async_copy(rows0_v, acc_sh.at[exv.at[0, 1]], sem, add=True)

        def body(jj, carry):
            j = jj * 2 + 1
            gather_unpack(j, rows1_v)
            pltpu.make_async_copy(
                rows0_v, acc_sh.at[exv.at[j - 1, 1]], sem).wait()
            pltpu.async_copy(rows1_v, acc_sh.at[exv.at[j, 1]], sem, add=True)
            gather_unpack(j + 1, rows0_v)
            pltpu.make_async_copy(
                rows1_v, acc_sh.at[exv.at[j, 1]], sem).wait()
            pltpu.async_copy(rows0_v, acc_sh.at[exv.at[j + 1, 1]], sem,
                             add=True)
            return carry

        lax.fori_loop(0, (nb - 1) // 2, body, 0)
        last = ((nb - 1) // 2) * 2
        if nb % 2 == 0:
            j = nb - 1
            gather_unpack(j, rows1_v)
            pltpu.make_async_copy(
                rows0_v, acc_sh.at[exv.at[last, 1]], sem).wait()
            pltpu.async_copy(rows1_v, acc_sh.at[exv.at[j, 1]], sem, add=True)
            pltpu.make_async_copy(
                rows1_v, acc_sh.at[exv.at[j, 1]], sem).wait()
        else:
            pltpu.make_async_copy(
                rows0_v, acc_sh.at[exv.at[last, 1]], sem).wait()
        plsc.subcore_barrier()
        for q in range(NQ):
            rq = r0 + q * qpt
            pltpu.sync_copy(acc_sh.at[pl.ds(rq, qpt)], dr_v)

            def conv(r, carry):
                for m in range(d // 32):
                    a = dr_v[r, pl.ds(32 * m, 16)]
                    b = dr_v[r, pl.ds(32 * m + 16, 16)]
                    drb_v[r, pl.ds(32 * m, 32)] = plsc.pack(
                        a, b, format=plsc.PackFormat.INTERLEAVED)
                return carry

            lax.fori_loop(0, qpt, conv, 0)
            pltpu.sync_copy(drb_v, out_h.at[c, pl.ds(rq, qpt)])

    return k(h_bf, edx)


def _tc_stage1(x_pad, w1, dp4, n_pad, hid):
    def body(x_ref, w_ref, dp_ref, ha_ref, hb_ref, dego_ref, degi_ref):
        do_ = jnp.maximum(dp_ref[0, 0] + dp_ref[1, 0], 1.0)
        di_ = jnp.maximum(dp_ref[0, 1] + dp_ref[1, 1], 1.0)
        dego = lax.rsqrt(do_)
        degi = lax.rsqrt(di_)
        dego_ref[...] = dego
        degi_ref[...] = degi
        h = jnp.dot(x_ref[...] * dego, w_ref[...],
                    preferred_element_type=jnp.float32)
        ha_ref[...] = h[:, : hid // 2]
        hb_ref[...] = h[:, hid // 2:]

    return pl.pallas_call(
        body,
        out_shape=(
            jax.ShapeDtypeStruct((n_pad, hid // 2), jnp.float32),
            jax.ShapeDtypeStruct((n_pad, hid // 2), jnp.float32),
            jax.ShapeDtypeStruct((n_pad, 1), jnp.float32),
            jax.ShapeDtypeStruct((n_pad, 1), jnp.float32),
        ),
    )(x_pad, w1, dp4)


def _tc_stage2(a1a, a1b, dego, degi, w2p, n_pad, hid, hid2):
    def body(aa_ref, ab_ref, dego_ref, degi_ref, w_ref, m_ref):
        def half(ref):
            a = ref[0].astype(jnp.float32) + ref[1].astype(jnp.float32)
            h = a * degi_ref[...]
            h = jnp.where(h >= 0, h, 0.01 * h)
            return h * dego_ref[...]

        ha = half(aa_ref)
        hb = half(ab_ref)
        m_ref[...] = (
            jnp.dot(ha, w_ref[: hid // 2], preferred_element_type=jnp.float32)
            + jnp.dot(hb, w_ref[hid // 2:], preferred_element_type=jnp.float32)
        )

    return pl.pallas_call(
        body,
        out_shape=jax.ShapeDtypeStruct((n_pad, hid2), jnp.float32),
    )(a1a, a1b, dego, degi, w2p)


def _tc_stage3(a2, degi, wlp, bl, wc, n):
    def body(a_ref, degi_ref, wl_ref, bl_ref, wc_ref, o_ref):
        a = a_ref[0].astype(jnp.float32) + a_ref[1].astype(jnp.float32)
        h = a * degi_ref[...]
        h = jnp.where(h >= 0, h, 0.01 * h)
        s = jnp.sum(h, axis=0, keepdims=True) * jnp.float32(1.0 / n)
        p = lax.dot_general(s, wl_ref[...], (((1,), (1,)), ((), ())),
                            preferred_element_type=jnp.float32) + bl_ref[...]
        o_ref[...] = lax.dot_general(p, wc_ref[...], (((1,), (1,)), ((), ())),
                                     preferred_element_type=jnp.float32)

    return pl.pallas_call(
        body,
        out_shape=jax.ShapeDtypeStruct((1, wc.shape[0]), jnp.float32),
    )(a2, degi, wlp, bl, wc)


def kernel(features, edge_index, W1, W2, Wl, bl, Wc):
    n, din = features.shape
    e = edge_index.shape[1]
    hid = W1.shape[1]
    hid2 = W2.shape[1]
    n_pad = _cdiv(n, NS * 16) * NS * 16
    nb = _cdiv(e, NW * B)
    e_pad = NW * nb * B

    src = edge_index[0].astype(jnp.int32)
    dst = edge_index[1].astype(jnp.int32)
    padi = jnp.full((e_pad - e,), n, jnp.int32)
    src3 = jnp.concatenate([src, padi]).reshape(NW, nb, B)
    dst3 = jnp.concatenate([dst, padi]).reshape(NW, nb, B)
    edx = jnp.stack([src3, dst3], axis=2)
    x_pad = jnp.concatenate(
        [features, jnp.zeros((n_pad - n, din), jnp.float32)])
    p64 = jnp.asarray(_pack_perm(hid2))
    w2p = W2[jnp.concatenate([p64, hid2 + p64])]
    wlp = Wl[:, p64]

    dp = _sc_degrees(edx, n_pad).reshape(NC, 2, n_pad, 1)
    h1a, h1b, dego, degi = _tc_stage1(x_pad, W1, dp, n_pad, hid)
    a1a = _sc_aggregate(h1a[:, p64].astype(jnp.bfloat16), edx, n_pad, hid2)
    a1b = _sc_aggregate(h1b[:, p64].astype(jnp.bfloat16), edx, n_pad, hid2)
    m2 = _tc_stage2(a1a, a1b, dego, degi, w2p, n_pad, hid, hid2)
    a2 = _sc_aggregate(m2[:, p64].astype(jnp.bfloat16), edx, n_pad, hid2)
    return _tc_stage3(a2, degi, wlp, bl, Wc, n)


# speedup vs baseline: 1.4681x; 1.0399x over previous
import functools

import numpy as np
import jax
import jax.numpy as jnp
from jax import lax
from jax.experimental import pallas as pl
from jax.experimental.pallas import tpu as pltpu
from jax.experimental.pallas import tpu_sc as plsc

NC = 2
NS = 16
NW = NC * NS
B = 128
NQ = 4


def _cdiv(a, b):
    return (a + b - 1) // b


def _pack_perm(d):
    perm = np.empty((d,), np.int32)
    for m in range(d // 32):
        for i in range(16):
            perm[32 * m + 2 * i] = 32 * m + i
            perm[32 * m + 2 * i + 1] = 32 * m + 16 + i
    return perm


def _zero_vmem(ref, rows, cols):
    z = jnp.zeros((16,), jnp.float32)

    def body(r, carry):
        for m in range(cols // 16):
            ref[r, pl.ds(16 * m, 16)] = z
        return carry

    lax.fori_loop(0, rows, body, 0)


def _sc_degrees(edx, n_pad):
    nb = edx.shape[1]
    rpt = n_pad // NS
    hpt = rpt // 2
    mesh = plsc.VectorSubcoreMesh(core_axis_name="c", subcore_axis_name="s")

    @functools.partial(
        pl.kernel,
        out_type=jax.ShapeDtypeStruct((NC * 2 * n_pad,), jnp.float32),
        mesh=mesh,
        scratch_types=[
            pltpu.VMEM((nb, 2, B), jnp.int32),
            pltpu.VMEM((B,), jnp.float32),
            pltpu.VMEM((hpt,), jnp.float32),
            pltpu.VMEM_SHARED((n_pad,), jnp.float32),
            pltpu.VMEM_SHARED((n_pad,), jnp.float32),
        ],
        compiler_params=pltpu.CompilerParams(use_tc_tiling_on_sc=False),
    )
    def k(edx_h, out_h, exv, ones_v, dr_v, dego_sh, degi_sh):
        c = lax.axis_index("c")
        s = lax.axis_index("s")
        wid = s * NC + c
        r0 = s * rpt
        def zb(r, carry):
            dr_v[pl.ds(16 * r, 16)] = jnp.zeros((16,), jnp.float32)
            return carry

        lax.fori_loop(0, hpt // 16, zb, 0)
        for i in range(B // 16):
            ones_v[pl.ds(i * 16, 16)] = jnp.ones((16,), jnp.float32)
        for sh in (dego_sh, degi_sh):
            for half in range(2):
                pltpu.sync_copy(dr_v, sh.at[pl.ds(r0 + half * hpt, hpt)])
        pltpu.sync_copy(edx_h.at[wid], exv)
        plsc.subcore_barrier()

        def body(j, carry):
            pltpu.sync_copy(ones_v, dego_sh.at[exv.at[j, 0]], add=True)
            pltpu.sync_copy(ones_v, degi_sh.at[exv.at[j, 1]], add=True)
            return carry

        lax.fori_loop(0, nb, body, 0)
        plsc.subcore_barrier()
        for t, sh in enumerate((dego_sh, degi_sh)):
            base = (c * 2 + t) * n_pad + r0
            for half in range(2):
                pltpu.sync_copy(sh.at[pl.ds(r0 + half * hpt, hpt)], dr_v)
                pltpu.sync_copy(dr_v, out_h.at[pl.ds(base + half * hpt, hpt)])

    return k(edx)


def _sc_aggregate(h_bf, edx, n_pad, d):
    nb = edx.shape[1]
    rpt = n_pad // NS
    qpt = rpt // NQ
    mesh = plsc.VectorSubcoreMesh(core_axis_name="c", subcore_axis_name="s")

    @functools.partial(
        pl.kernel,
        out_type=jax.ShapeDtypeStruct((NC, n_pad, d), jnp.bfloat16),
        mesh=mesh,
        scratch_types=[
            pltpu.VMEM((nb, 2, B), jnp.int32),
            pltpu.VMEM((B, d), jnp.bfloat16),
            pltpu.VMEM((B, d), jnp.float32),
            pltpu.VMEM((B, d), jnp.float32),
            pltpu.VMEM((qpt, d), jnp.float32),
            pltpu.VMEM((qpt, d), jnp.bfloat16),
            pltpu.VMEM_SHARED((n_pad, d), jnp.bfloat16),
            pltpu.VMEM_SHARED((n_pad, d), jnp.float32),
            pltpu.SemaphoreType.DMA,
        ],
        compiler_params=pltpu.CompilerParams(
            use_tc_tiling_on_sc=False,
            needs_layout_passes=False,
        ),
    )
    def k(h_h, edx_h, out_h, exv, rbf_v, rows0_v, rows1_v, dr_v, drb_v,
          tab_sh, acc_sh, sem):
        c = lax.axis_index("c")
        s = lax.axis_index("s")
        wid = s * NC + c
        r0 = s * rpt
        for q in range(NQ):
            rq = r0 + q * qpt
            pltpu.sync_copy(h_h.at[pl.ds(rq, qpt)], drb_v)
            pltpu.sync_copy(drb_v, tab_sh.at[pl.ds(rq, qpt)])
        _zero_vmem(dr_v, qpt, d)
        for q in range(NQ):
            pltpu.sync_copy(dr_v, acc_sh.at[pl.ds(r0 + q * qpt, qpt)])
        pltpu.sync_copy(edx_h.at[wid], exv)
        plsc.subcore_barrier()

        def gather_unpack(j, rows_v):
            pltpu.sync_copy(tab_sh.at[exv.at[j, 0]], rbf_v)

            @plsc.parallel_loop(0, B, unroll=8)
            def conv(r):
                for m in range(d // 32):
                    u = rbf_v[r, pl.ds(32 * m, 32)]
                    a, b = plsc.unpack(u, format=plsc.PackFormat.INTERLEAVED)
                    rows_v[r, pl.ds(32 * m, 16)] = a
                    rows_v[r, pl.ds(32 * m + 16, 16)] = b

        gather_unpack(0, rows0_v)
        pltpu.async_copy(rows0_v, acc_sh.at[exv.at[0, 1]], sem, add=True)

        def body(jj, carry):
            j = jj * 2 + 1
            gather_unpack(j, rows1_v)
            pltpu.make_async_copy(
                rows0_v, acc_sh.at[exv.at[j - 1, 1]], sem).wait()
            pltpu.async_copy(rows1_v, acc_sh.at[exv.at[j, 1]], sem, add=True)
            gather_unpack(j + 1, rows0_v)
            pltpu.make_async_copy(
                rows1_v, acc_sh.at[exv.at[j, 1]], sem).wait()
            pltpu.async_copy(rows0_v, acc_sh.at[exv.at[j + 1, 1]], sem,
                             add=True)
            return carry

        lax.fori_loop(0, (nb - 1) // 2, body, 0)
        last = ((nb - 1) // 2) * 2
        if nb % 2 == 0:
            j = nb - 1
            gather_unpack(j, rows1_v)
            pltpu.make_async_copy(
                rows0_v, acc_sh.at[exv.at[last, 1]], sem).wait()
            pltpu.async_copy(rows1_v, acc_sh.at[exv.at[j, 1]], sem, add=True)
            pltpu.make_async_copy(
                rows1_v, acc_sh.at[exv.at[j, 1]], sem).wait()
        else:
            pltpu.make_async_copy(
                rows0_v, acc_sh.at[exv.at[last, 1]], sem).wait()
        plsc.subcore_barrier()
        for q in range(NQ):
            rq = r0 + q * qpt
            pltpu.sync_copy(acc_sh.at[pl.ds(rq, qpt)], dr_v)

            def conv(r, carry):
                for m in range(d // 32):
                    a = dr_v[r, pl.ds(32 * m, 16)]
                    b = dr_v[r, pl.ds(32 * m + 16, 16)]
                    drb_v[r, pl.ds(32 * m, 32)] = plsc.pack(
                        a, b, format=plsc.PackFormat.INTERLEAVED)
                return carry

            lax.fori_loop(0, qpt, conv, 0)
            pltpu.sync_copy(drb_v, out_h.at[c, pl.ds(rq, qpt)])

    return k(h_bf, edx)


def _tc_matmul1(x_pad, w1p, n_pad, hid):
    def body(x_ref, w_ref, p_ref):
        p_ref[...] = jnp.dot(x_ref[...], w_ref[...],
                             preferred_element_type=jnp.float32)

    return pl.pallas_call(
        body,
        out_shape=jax.ShapeDtypeStruct((n_pad, hid), jnp.float32),
    )(x_pad, w1p)


def _tc_stage1(p1, dp4, n_pad, hid):
    def body(p_ref, dp_ref, ha_ref, hb_ref, dego_ref, degi_ref):
        do_ = jnp.maximum(dp_ref[0, 0] + dp_ref[1, 0], 1.0)
        di_ = jnp.maximum(dp_ref[0, 1] + dp_ref[1, 1], 1.0)
        dego = lax.rsqrt(do_)
        degi = lax.rsqrt(di_)
        dego_ref[...] = dego
        degi_ref[...] = degi
        h = p_ref[...] * dego
        ha_ref[...] = h[:, : hid // 2].astype(jnp.bfloat16)
        hb_ref[...] = h[:, hid // 2:].astype(jnp.bfloat16)

    return pl.pallas_call(
        body,
        out_shape=(
            jax.ShapeDtypeStruct((n_pad, hid // 2), jnp.bfloat16),
            jax.ShapeDtypeStruct((n_pad, hid // 2), jnp.bfloat16),
            jax.ShapeDtypeStruct((n_pad, 1), jnp.float32),
            jax.ShapeDtypeStruct((n_pad, 1), jnp.float32),
        ),
    )(p1, dp4)


def _tc_stage2(a1a, a1b, dego, degi, w2p, n_pad, hid, hid2):
    def body(aa_ref, ab_ref, dego_ref, degi_ref, w_ref, m_ref):
        def half(ref):
            a = ref[0].astype(jnp.float32) + ref[1].astype(jnp.float32)
            h = a * degi_ref[...]
            h = jnp.where(h >= 0, h, 0.01 * h)
            return h * dego_ref[...]

        ha = half(aa_ref)
        hb = half(ab_ref)
        m_ref[...] = (
            jnp.dot(ha, w_ref[: hid // 2], preferred_element_type=jnp.float32)
            + jnp.dot(hb, w_ref[hid // 2:], preferred_element_type=jnp.float32)
        ).astype(jnp.bfloat16)

    return pl.pallas_call(
        body,
        out_shape=jax.ShapeDtypeStruct((n_pad, hid2), jnp.bfloat16),
    )(a1a, a1b, dego, degi, w2p)


def _tc_stage3(a2, degi, wlp, bl, wc, n):
    def body(a_ref, degi_ref, wl_ref, bl_ref, wc_ref, o_ref):
        a = a_ref[0].astype(jnp.float32) + a_ref[1].astype(jnp.float32)
        h = a * degi_ref[...]
        h = jnp.where(h >= 0, h, 0.01 * h)
        s = jnp.sum(h, axis=0, keepdims=True) * jnp.float32(1.0 / n)
        p = lax.dot_general(s, wl_ref[...], (((1,), (1,)), ((), ())),
                            preferred_element_type=jnp.float32) + bl_ref[...]
        o_ref[...] = lax.dot_general(p, wc_ref[...], (((1,), (1,)), ((), ())),
                                     preferred_element_type=jnp.float32)

    return pl.pallas_call(
        body,
        out_shape=jax.ShapeDtypeStruct((1, wc.shape[0]), jnp.float32),
    )(a2, degi, wlp, bl, wc)


def kernel(features, edge_index, W1, W2, Wl, bl, Wc):
    n, din = features.shape
    e = edge_index.shape[1]
    hid = W1.shape[1]
    hid2 = W2.shape[1]
    n_pad = _cdiv(n, NS * 16) * NS * 16
    nb = _cdiv(e, NW * B)
    e_pad = NW * nb * B

    src = edge_index[0].astype(jnp.int32)
    dst = edge_index[1].astype(jnp.int32)
    padi = jnp.full((e_pad - e,), n, jnp.int32)
    src3 = jnp.concatenate([src, padi]).reshape(NW, nb, B)
    dst3 = jnp.concatenate([dst, padi]).reshape(NW, nb, B)
    edx = jnp.stack([src3, dst3], axis=2)
    x_pad = jnp.concatenate(
        [features, jnp.zeros((n_pad - n, din), jnp.float32)])
    p64 = jnp.asarray(_pack_perm(hid2))
    p128 = jnp.concatenate([p64, hid2 + p64])
    w1p = W1[:, p128]
    w2p = W2[p128][:, p64]
    wlp = Wl[:, p64]

    dp = _sc_degrees(edx, n_pad).reshape(NC, 2, n_pad, 1)
    p1 = _tc_matmul1(x_pad, w1p, n_pad, hid)
    h1a, h1b, dego, degi = _tc_stage1(p1, dp, n_pad, hid)
    a1a = _sc_aggregate(h1a, edx, n_pad, hid2)
    a1b = _sc_aggregate(h1b, edx, n_pad, hid2)
    m2 = _tc_stage2(a1a, a1b, dego, degi, w2p, n_pad, hid, hid2)
    a2 = _sc_aggregate(m2, edx, n_pad, hid2)
    return _tc_stage3(a2, degi, wlp, bl, Wc, n)
